# baseline (speedup 1.0000x reference)
import jax
import jax.numpy as jnp
from jax import lax
from jax.experimental import pallas as pl
from jax.experimental.pallas import tpu as pltpu
from jax.experimental.pallas import tpu_sc as plsc

_B, _T, _C = 4, 2048, 1024
_NC, _NS = 2, 16
_NW = _NC * _NS
_RPW = _T // _NW
_CH = 8
_STEPS = _RPW // _CH
_LANES = 16


def _pe_add_body(x_hbm, pe_hbm, out_hbm, pe_v, x_v):
    wid = lax.axis_index("s") * _NC + lax.axis_index("c")
    pbase = wid * _RPW

    def step(s, carry):
        r0 = pbase + s * _CH
        pltpu.sync_copy(pe_hbm.at[pl.ds(r0, _CH)], pe_v)
        for b in range(_B):
            pltpu.sync_copy(x_hbm.at[b, pl.ds(r0, _CH)], x_v)
            for r in range(_CH):
                def body(j, c):
                    sl = pl.ds(j * _LANES, _LANES)
                    x_v[r, sl] = x_v[r, sl] + pe_v[r, sl]
                    return c
                lax.fori_loop(0, _C // _LANES, body, 0)
            pltpu.sync_copy(x_v, out_hbm.at[b, pl.ds(r0, _CH)])
        return carry

    lax.fori_loop(0, _STEPS, step, 0)


def kernel(x, pos_emb):
    mesh = plsc.VectorSubcoreMesh(core_axis_name="c", subcore_axis_name="s")
    f = pl.kernel(
        _pe_add_body,
        mesh=mesh,
        out_type=jax.ShapeDtypeStruct((_B, _T, _C), jnp.float32),
        scratch_types=[
            pltpu.VMEM((_CH, _C), jnp.float32),
            pltpu.VMEM((_CH, _C), jnp.float32),
        ],
    )
    return f(x, pos_emb)


# speedup vs baseline: 2.6193x; 2.6193x over previous
import jax
import jax.numpy as jnp
from jax import lax
from jax.experimental import pallas as pl
from jax.experimental.pallas import tpu as pltpu
from jax.experimental.pallas import tpu_sc as plsc

_B, _T, _C = 4, 2048, 1024
_NC, _NS = 2, 16
_NW = _NC * _NS
_RPW = _T // _NW
_CH = 8
_STEPS = _RPW // _CH
_NBUF = 3
_NCHUNK = _STEPS * _B


def _pe_add_body(x_hbm, pe_hbm, out_hbm, xi0, xi1, xi2, xo0, xo1, xo2,
                 pv0, pv1, ld0, ld1, ld2, st0, st1, st2, pes0, pes1):
    xin = [xi0, xi1, xi2]
    xout = [xo0, xo1, xo2]
    pev = [pv0, pv1]
    ldsems = [ld0, ld1, ld2]
    stsems = [st0, st1, st2]
    pesems = [pes0, pes1]
    wid = lax.axis_index("s") * _NC + lax.axis_index("c")
    rbase = wid * _RPW

    ld_desc = {}
    st_desc = {}
    pe_desc = {}

    def rows(k):
        s, b = divmod(k, _B)
        return b, pl.ds(rbase + s * _CH, _CH)

    def issue_load(k):
        buf = k % _NBUF
        b, sl = rows(k)
        ld_desc[buf] = pltpu.async_copy(x_hbm.at[b, sl], xin[buf],
                                        ldsems[buf])

    def issue_pe(s):
        pb = s % 2
        pe_desc[pb] = pltpu.async_copy(
            pe_hbm.at[pl.ds(rbase + s * _CH, _CH)], pev[pb], pesems[pb])

    issue_pe(0)
    issue_pe(1)
    for k in range(_NBUF):
        issue_load(k)

    for k in range(_NCHUNK):
        s, b = divmod(k, _B)
        buf = k % _NBUF
        pb = s % 2
        if b == 0:
            if 1 <= s < _STEPS - 1:
                issue_pe(s + 1)
            pe_desc[pb].wait()
        ld_desc[buf].wait()
        if k >= _NBUF:
            st_desc[buf].wait()
        xi = xin[buf]
        xo = xout[buf]
        pv = pev[pb]

        @plsc.parallel_loop(0, _C, step=16, unroll=2)
        def _(j):
            sl = pl.ds(j, 16)
            for r in range(_CH):
                xo[r, sl] = xi[r, sl] + pv[r, sl]

        _, osl = rows(k)
        st_desc[buf] = pltpu.async_copy(xo, out_hbm.at[b, osl], stsems[buf])
        if k + _NBUF < _NCHUNK:
            issue_load(k + _NBUF)

    for k in range(_NCHUNK - _NBUF, _NCHUNK):
        st_desc[k % _NBUF].wait()


def kernel(x, pos_emb):
    mesh = plsc.VectorSubcoreMesh(core_axis_name="c", subcore_axis_name="s")
    f = pl.kernel(
        _pe_add_body,
        mesh=mesh,
        out_type=jax.ShapeDtypeStruct((_B, _T, _C), jnp.float32),
        scratch_types=[
            pltpu.VMEM((_CH, _C), jnp.float32),
            pltpu.VMEM((_CH, _C), jnp.float32),
            pltpu.VMEM((_CH, _C), jnp.float32),
            pltpu.VMEM((_CH, _C), jnp.float32),
            pltpu.VMEM((_CH, _C), jnp.float32),
            pltpu.VMEM((_CH, _C), jnp.float32),
            pltpu.VMEM((_CH, _C), jnp.float32),
            pltpu.VMEM((_CH, _C), jnp.float32),
            pltpu.SemaphoreType.DMA,
            pltpu.SemaphoreType.DMA,
            pltpu.SemaphoreType.DMA,
            pltpu.SemaphoreType.DMA,
            pltpu.SemaphoreType.DMA,
            pltpu.SemaphoreType.DMA,
            pltpu.SemaphoreType.DMA,
            pltpu.SemaphoreType.DMA,
        ],
    )
    return f(x, pos_emb)
